# baseline (speedup 1.0000x reference)
import jax
import jax.numpy as jnp
from jax import lax
from jax.experimental import pallas as pl
from jax.experimental.pallas import tpu as pltpu
from jax.experimental.pallas import tpu_sc as plsc

D_MODEL = 768
SEQ = 8192
BATCH = 4

NC = 2
NS = 16
NW = NC * NS

SEQ_SC = 4096
ROWS_PER_W = SEQ_SC // NW
CH = 8
N_CHUNKS = ROWS_PER_W // CH
CHW = CH * D_MODEL
G = 8
N_GROUPS = CHW // (16 * G)

BS = 1024
RB0 = SEQ_SC // BS
NRB = (SEQ - SEQ_SC) // BS


def _sc_body(x_hbm, pos_hbm, dummy_hbm, out_hbm,
             xb0, xb1, ob0, ob1, pb0, pb1,
             sx0, sx1, so0, so1, sp0, sp1):
    xb = (xb0, xb1)
    ob = (ob0, ob1)
    pb = (pb0, pb1)
    sx = (sx0, sx1)
    so = (so0, so1)
    sp = (sp0, sp1)

    w = lax.axis_index("s") * NC + lax.axis_index("c")
    base = w * ROWS_PER_W

    def in_copies(c, slot):
        r0 = base + c * CH
        copies = [pltpu.make_async_copy(
            pos_hbm.at[r0 + r, :],
            pb[slot].at[pl.ds(r * D_MODEL, D_MODEL)], sp[slot])
            for r in range(CH)]
        copies += [pltpu.make_async_copy(
            x_hbm.at[b, r0 + r, :],
            xb[slot].at[pl.ds(b * CHW + r * D_MODEL, D_MODEL)], sx[slot])
            for b in range(BATCH) for r in range(CH)]
        return copies

    def out_copies(c, slot):
        r0 = base + c * CH
        return [pltpu.make_async_copy(
            ob[slot].at[pl.ds(b * CHW + r * D_MODEL, D_MODEL)],
            out_hbm.at[b, r0 + r, :], so[slot])
            for b in range(BATCH) for r in range(CH)]

    def start_in(c, slot):
        for cp in in_copies(c, slot):
            cp.start()

    def wait_in(c, slot):
        pltpu.make_async_copy(dummy_hbm.at[pl.ds(0, CHW)],
                              pb[slot], sp[slot]).wait()
        pltpu.make_async_copy(dummy_hbm, xb[slot], sx[slot]).wait()

    def start_out(c, slot):
        for cp in out_copies(c, slot):
            cp.start()

    def wait_out(c, slot):
        pltpu.make_async_copy(ob[slot], dummy_hbm, so[slot]).wait()

    def compute(slot):
        xs, os_, ps = xb[slot], ob[slot], pb[slot]

        @plsc.parallel_loop(0, N_GROUPS)
        def _(i):
            gbase = i * (16 * G)
            pos_vecs = [ps[pl.ds(gbase + k * 16, 16)] for k in range(G)]
            for b in range(BATCH):
                for k in range(G):
                    sl = pl.ds(b * CHW + gbase + k * 16, 16)
                    os_[sl] = xs[sl] + pos_vecs[k]

    start_in(0, 0)
    start_in(1, 1)
    for c in (0, 1):
        wait_in(c, c)
        compute(c)
        start_out(c, c)
        start_in(c + 2, c)

    @pl.loop(2, N_CHUNKS, step=2)
    def _(c0):
        for d in range(2):
            c = c0 + d
            wait_in(c, d)
            wait_out(c - 2, d)
            compute(d)
            start_out(c, d)

            @pl.when(c + 2 < N_CHUNKS)
            def _():
                start_in(c + 2, d)

    wait_out(N_CHUNKS - 2, 0)
    wait_out(N_CHUNKS - 1, 1)


def _tc_body(x_ref, pos_ref, sc_ref, out_ref):
    del sc_ref
    out_ref[...] = x_ref[...] + pos_ref[...]


@jax.jit
def kernel(x, pos_embedding):
    seq = x.shape[1]
    pos = pos_embedding[:seq]
    dummy = jnp.zeros((BATCH * CHW,), jnp.float32)
    mesh = plsc.VectorSubcoreMesh(core_axis_name="c", subcore_axis_name="s")
    sc_out = pl.kernel(
        _sc_body,
        mesh=mesh,
        out_type=jax.ShapeDtypeStruct(x.shape, jnp.float32),
        scratch_types=(
            [pltpu.VMEM((BATCH * CHW,), jnp.float32)] * 4
            + [pltpu.VMEM((CHW,), jnp.float32)] * 2
            + [pltpu.SemaphoreType.DMA] * 6
        ),
    )(x, pos, dummy)

    return pl.pallas_call(
        _tc_body,
        grid=(NRB,),
        in_specs=[
            pl.BlockSpec((BATCH, BS, D_MODEL), lambda rb: (0, RB0 + rb, 0)),
            pl.BlockSpec((BS, D_MODEL), lambda rb: (RB0 + rb, 0)),
            pl.BlockSpec(memory_space=pl.ANY),
        ],
        out_specs=pl.BlockSpec((BATCH, BS, D_MODEL), lambda rb: (0, RB0 + rb, 0)),
        out_shape=jax.ShapeDtypeStruct(x.shape, jnp.float32),
        input_output_aliases={2: 0},
    )(x, pos, sc_out)


# speedup vs baseline: 1.0147x; 1.0147x over previous
import jax
import jax.numpy as jnp
from jax import lax
from jax.experimental import pallas as pl
from jax.experimental.pallas import tpu as pltpu
from jax.experimental.pallas import tpu_sc as plsc

D_MODEL = 768
SEQ = 8192
BATCH = 4

NC = 2
NS = 16
NW = NC * NS

SEQ_SC = 3072
ROWS_PER_W = SEQ_SC // NW
CH = 8
N_CHUNKS = ROWS_PER_W // CH
CHW = CH * D_MODEL
G = 8
N_GROUPS = CHW // (16 * G)

BS = 512
RB0 = SEQ_SC // BS
NRB = (SEQ - SEQ_SC) // BS


def _sc_body(x_hbm, pos_hbm, dummy_hbm, out_hbm,
             xb0, xb1, ob0, ob1, pb0, pb1,
             sx0, sx1, so0, so1, sp0, sp1):
    xb = (xb0, xb1)
    ob = (ob0, ob1)
    pb = (pb0, pb1)
    sx = (sx0, sx1)
    so = (so0, so1)
    sp = (sp0, sp1)

    w = lax.axis_index("s") * NC + lax.axis_index("c")
    base = w * ROWS_PER_W

    def in_copies(c, slot):
        r0 = base + c * CH
        copies = [pltpu.make_async_copy(
            pos_hbm.at[r0 + r, :],
            pb[slot].at[pl.ds(r * D_MODEL, D_MODEL)], sp[slot])
            for r in range(CH)]
        copies += [pltpu.make_async_copy(
            x_hbm.at[b, r0 + r, :],
            xb[slot].at[pl.ds(b * CHW + r * D_MODEL, D_MODEL)], sx[slot])
            for b in range(BATCH) for r in range(CH)]
        return copies

    def out_copies(c, slot):
        r0 = base + c * CH
        return [pltpu.make_async_copy(
            ob[slot].at[pl.ds(b * CHW + r * D_MODEL, D_MODEL)],
            out_hbm.at[b, r0 + r, :], so[slot])
            for b in range(BATCH) for r in range(CH)]

    def start_in(c, slot):
        for cp in in_copies(c, slot):
            cp.start()

    def wait_in(c, slot):
        pltpu.make_async_copy(dummy_hbm.at[pl.ds(0, CHW)],
                              pb[slot], sp[slot]).wait()
        pltpu.make_async_copy(dummy_hbm, xb[slot], sx[slot]).wait()

    def start_out(c, slot):
        for cp in out_copies(c, slot):
            cp.start()

    def wait_out(c, slot):
        pltpu.make_async_copy(ob[slot], dummy_hbm, so[slot]).wait()

    def compute(slot):
        xs, os_, ps = xb[slot], ob[slot], pb[slot]

        @plsc.parallel_loop(0, N_GROUPS)
        def _(i):
            gbase = i * (16 * G)
            pos_vecs = [ps[pl.ds(gbase + k * 16, 16)] for k in range(G)]
            for b in range(BATCH):
                for k in range(G):
                    sl = pl.ds(b * CHW + gbase + k * 16, 16)
                    os_[sl] = xs[sl] + pos_vecs[k]

    start_in(0, 0)
    start_in(1, 1)
    for c in (0, 1):
        wait_in(c, c)
        compute(c)
        start_out(c, c)
        start_in(c + 2, c)

    @pl.loop(2, N_CHUNKS, step=2)
    def _(c0):
        for d in range(2):
            c = c0 + d
            wait_in(c, d)
            wait_out(c - 2, d)
            compute(d)
            start_out(c, d)

            @pl.when(c + 2 < N_CHUNKS)
            def _():
                start_in(c + 2, d)

    wait_out(N_CHUNKS - 2, 0)
    wait_out(N_CHUNKS - 1, 1)


def _tc_body(x_ref, pos_ref, sc_ref, out_ref):
    del sc_ref
    out_ref[...] = x_ref[...] + pos_ref[...]


@jax.jit
def kernel(x, pos_embedding):
    seq = x.shape[1]
    pos = pos_embedding[:seq]
    dummy = jnp.zeros((BATCH * CHW,), jnp.float32)
    mesh = plsc.VectorSubcoreMesh(core_axis_name="c", subcore_axis_name="s")
    sc_out = pl.kernel(
        _sc_body,
        mesh=mesh,
        out_type=jax.ShapeDtypeStruct(x.shape, jnp.float32),
        scratch_types=(
            [pltpu.VMEM((BATCH * CHW,), jnp.float32)] * 4
            + [pltpu.VMEM((CHW,), jnp.float32)] * 2
            + [pltpu.SemaphoreType.DMA] * 6
        ),
    )(x, pos, dummy)

    return pl.pallas_call(
        _tc_body,
        grid=(NRB,),
        in_specs=[
            pl.BlockSpec((BATCH, BS, D_MODEL), lambda rb: (0, RB0 + rb, 0)),
            pl.BlockSpec((BS, D_MODEL), lambda rb: (RB0 + rb, 0)),
            pl.BlockSpec(memory_space=pl.ANY),
        ],
        out_specs=pl.BlockSpec((BATCH, BS, D_MODEL), lambda rb: (0, RB0 + rb, 0)),
        out_shape=jax.ShapeDtypeStruct(x.shape, jnp.float32),
        input_output_aliases={2: 0},
    )(x, pos, sc_out)
